# baseline (speedup 1.0000x reference)
import functools

import jax
import jax.numpy as jnp
from jax import lax
from jax.experimental import pallas as pl
from jax.experimental.pallas import tpu as pltpu
from jax.experimental.pallas import tpu_sc as plsc

H = 768
F = 2048
E = 64
N = 4096
B_TOK = 256
NB = N // B_TOK
G = NB + E - 1

NW = 32
BPW = N // NW


def _router_body(x_ref, wr_ref, out_ref):
    logits = lax.dot_general(
        x_ref[...], wr_ref[...], (((1,), (1,)), ((), ())),
        preferred_element_type=jnp.float32)
    m = jnp.max(logits, axis=1, keepdims=True)
    col = lax.broadcasted_iota(jnp.int32, logits.shape, 1)
    eid = jnp.min(jnp.where(logits >= m, col, E), axis=1).astype(jnp.int32)
    out_ref[...] = eid.reshape(out_ref.shape)


def _router(x_flat, Wr):
    out = pl.pallas_call(
        _router_body,
        grid=(8,),
        in_specs=[
            pl.BlockSpec((512, H), lambda t: (t, 0)),
            pl.BlockSpec((E, H), lambda t: (0, 0)),
        ],
        out_specs=pl.BlockSpec((1, 4, 128), lambda t: (t, 0, 0)),
        out_shape=jax.ShapeDtypeStruct((8, 4, 128), jnp.int32),
    )(x_flat, Wr)
    return out.reshape(N)


@functools.lru_cache(maxsize=None)
def _sc_gather_kernel():
    mesh = plsc.VectorSubcoreMesh(core_axis_name="c", subcore_axis_name="s")

    @functools.partial(
        pl.kernel,
        mesh=mesh,
        out_type=jax.ShapeDtypeStruct((N, H), jnp.float32),
        scratch_types=[
            pltpu.VMEM((BPW,), jnp.int32),
            pltpu.VMEM((BPW, H), jnp.float32),
            pltpu.SemaphoreType.DMA,
        ],
    )
    def gather(x_hbm, idx_hbm, out_hbm, idx_v, rows_v, sem):
        wid = lax.axis_index("s") * 2 + lax.axis_index("c")
        base = wid * BPW
        pltpu.sync_copy(idx_hbm.at[pl.ds(base, BPW)], idx_v)
        pltpu.async_copy(x_hbm.at[idx_v], rows_v, sem).wait()
        pltpu.sync_copy(rows_v, out_hbm.at[pl.ds(base, BPW)])

    return gather


@functools.lru_cache(maxsize=None)
def _sc_scatter_kernel():
    mesh = plsc.VectorSubcoreMesh(core_axis_name="c", subcore_axis_name="s")

    @functools.partial(
        pl.kernel,
        mesh=mesh,
        out_type=jax.ShapeDtypeStruct((N, H), jnp.float32),
        scratch_types=[
            pltpu.VMEM((BPW,), jnp.int32),
            pltpu.VMEM((BPW, H), jnp.float32),
            pltpu.SemaphoreType.DMA,
        ],
    )
    def scatter(y_hbm, idx_hbm, out_hbm, idx_v, rows_v, sem):
        wid = lax.axis_index("s") * 2 + lax.axis_index("c")
        base = wid * BPW
        pltpu.sync_copy(idx_hbm.at[pl.ds(base, BPW)], idx_v)
        pltpu.sync_copy(y_hbm.at[pl.ds(base, BPW)], rows_v)
        pltpu.async_copy(rows_v, out_hbm.at[idx_v], sem).wait()

    return scatter


NF = 2
FB = F // NF


def _gmm_body(tb_ref, eb_ref, lo_ref, hi_ref, first_ref,
              x_ref, wg_ref, wu_ref, wd_ref, out_ref):
    w = pl.program_id(0)
    j = pl.program_id(1)

    @pl.when((first_ref[w] == 1) & (j == 0))
    def _():
        out_ref[...] = jnp.zeros_like(out_ref)

    lo = lo_ref[w]
    hi = hi_ref[w]

    @pl.when(hi > lo)
    def _():
        rows = lax.broadcasted_iota(jnp.int32, (B_TOK, 1), 0)
        msk = ((rows >= lo) & (rows < hi)).astype(jnp.float32)
        xm = x_ref[...] * msk
        g = lax.dot_general(xm, wg_ref[0], (((1,), (1,)), ((), ())),
                            preferred_element_type=jnp.float32)
        u = lax.dot_general(xm, wu_ref[0], (((1,), (1,)), ((), ())),
                            preferred_element_type=jnp.float32)
        hdn = (g * jax.nn.sigmoid(g)) * u
        y = lax.dot_general(hdn, wd_ref[0], (((1,), (1,)), ((), ())),
                            preferred_element_type=jnp.float32)
        out_ref[...] += y


def _gmm(x_sorted, Wg, Wu, Wd, tb, eb, lo, hi, first):
    grid_spec = pltpu.PrefetchScalarGridSpec(
        num_scalar_prefetch=5,
        grid=(G, NF),
        in_specs=[
            pl.BlockSpec((B_TOK, H), lambda w, j, tb, eb, lo, hi, first: (tb[w], 0)),
            pl.BlockSpec((1, FB, H), lambda w, j, tb, eb, lo, hi, first: (eb[w], j, 0)),
            pl.BlockSpec((1, FB, H), lambda w, j, tb, eb, lo, hi, first: (eb[w], j, 0)),
            pl.BlockSpec((1, H, FB), lambda w, j, tb, eb, lo, hi, first: (eb[w], 0, j)),
        ],
        out_specs=pl.BlockSpec((B_TOK, H), lambda w, j, tb, eb, lo, hi, first: (tb[w], 0)),
    )
    return pl.pallas_call(
        _gmm_body,
        grid_spec=grid_spec,
        out_shape=jax.ShapeDtypeStruct((N, H), jnp.float32),
        compiler_params=pltpu.CompilerParams(
            dimension_semantics=("arbitrary", "arbitrary")),
    )(tb, eb, lo, hi, first, x_sorted, Wg, Wu, Wd)


def _schedule(eid):
    counts = jnp.zeros((E,), jnp.int32).at[eid].add(1)
    ends = jnp.cumsum(counts)
    starts = ends - counts
    nonempty = counts > 0
    fb = jnp.where(nonempty, starts // B_TOK, 0)
    lb = jnp.where(nonempty, (ends - 1) // B_TOK, -1)
    nb = jnp.where(nonempty, lb - fb + 1, 0).astype(jnp.int32)
    cnb = jnp.cumsum(nb)
    total = cnb[E - 1]
    w = jnp.arange(G, dtype=jnp.int32)
    wc = jnp.minimum(w, total - 1)
    e_of = jnp.searchsorted(cnb, wc, side="right").astype(jnp.int32)
    ws = cnb[e_of] - nb[e_of]
    tb = (fb[e_of] + (wc - ws)).astype(jnp.int32)
    lo = jnp.maximum(starts[e_of] - tb * B_TOK, 0)
    hi = jnp.minimum(ends[e_of] - tb * B_TOK, B_TOK)
    valid = w < total
    lo = jnp.where(valid, lo, 0).astype(jnp.int32)
    hi = jnp.where(valid, hi, 0).astype(jnp.int32)
    first = jnp.concatenate(
        [jnp.ones((1,), jnp.int32), (tb[1:] != tb[:-1]).astype(jnp.int32)])
    return tb, e_of, lo, hi, first


def kernel(x, Wr, Wg, Wu, Wd):
    b, s, d = x.shape
    x_flat = x.reshape(N, H)
    eid = _router(x_flat, Wr)
    sort_idx = jnp.argsort(eid).astype(jnp.int32)
    tb, eb, lo, hi, first = _schedule(eid)
    x_sorted = _sc_gather_kernel()(x_flat, sort_idx)
    y_sorted = _gmm(x_sorted, Wg, Wu, Wd, tb, eb, lo, hi, first)
    out = _sc_scatter_kernel()(y_sorted, sort_idx)
    return out.reshape(b, s, d)


# speedup vs baseline: 1.0688x; 1.0688x over previous
import functools

import jax
import jax.numpy as jnp
from jax import lax
from jax.experimental import pallas as pl
from jax.experimental.pallas import tpu as pltpu
from jax.experimental.pallas import tpu_sc as plsc

H = 768
F = 2048
E = 64
N = 4096
B_TOK = 256
NB = N // B_TOK
G = NB + E - 1

NW = 32
BPW = N // NW


def _router_body(x_ref, wr_ref, out_ref, cnt_ref):
    t = pl.program_id(0)
    logits = lax.dot_general(
        x_ref[...], wr_ref[...], (((1,), (1,)), ((), ())),
        preferred_element_type=jnp.float32)
    m = jnp.max(logits, axis=1, keepdims=True)
    col = lax.broadcasted_iota(jnp.int32, logits.shape, 1)
    eid = jnp.min(jnp.where(logits >= m, col, E), axis=1).astype(jnp.int32)
    out_ref[...] = eid.reshape(out_ref.shape)

    @pl.when(t == 0)
    def _():
        cnt_ref[...] = jnp.zeros_like(cnt_ref)

    onehot = (col == eid[:, None]).astype(jnp.int32)
    cnt_ref[...] += jnp.sum(onehot, axis=0, keepdims=True)


def _router(x_flat, Wr):
    out, cnt = pl.pallas_call(
        _router_body,
        grid=(8,),
        in_specs=[
            pl.BlockSpec((512, H), lambda t: (t, 0)),
            pl.BlockSpec((E, H), lambda t: (0, 0)),
        ],
        out_specs=[
            pl.BlockSpec((1, 4, 128), lambda t: (t, 0, 0)),
            pl.BlockSpec((1, E), lambda t: (0, 0)),
        ],
        out_shape=[
            jax.ShapeDtypeStruct((8, 4, 128), jnp.int32),
            jax.ShapeDtypeStruct((1, E), jnp.int32),
        ],
        compiler_params=pltpu.CompilerParams(
            dimension_semantics=("arbitrary",)),
    )(x_flat, Wr)
    return out.reshape(N), cnt.reshape(E)


@functools.lru_cache(maxsize=None)
def _sc_gather_kernel():
    mesh = plsc.VectorSubcoreMesh(core_axis_name="c", subcore_axis_name="s")

    @functools.partial(
        pl.kernel,
        mesh=mesh,
        out_type=jax.ShapeDtypeStruct((N, H), jnp.float32),
        scratch_types=[
            pltpu.VMEM((BPW,), jnp.int32),
            pltpu.VMEM((BPW, H), jnp.float32),
            pltpu.SemaphoreType.DMA,
        ],
    )
    def gather(x_hbm, idx_hbm, out_hbm, idx_v, rows_v, sem):
        wid = lax.axis_index("s") * 2 + lax.axis_index("c")
        base = wid * BPW
        pltpu.sync_copy(idx_hbm.at[pl.ds(base, BPW)], idx_v)
        pltpu.async_copy(x_hbm.at[idx_v], rows_v, sem).wait()
        pltpu.sync_copy(rows_v, out_hbm.at[pl.ds(base, BPW)])

    return gather


@functools.lru_cache(maxsize=None)
def _sc_scatter_kernel():
    mesh = plsc.VectorSubcoreMesh(core_axis_name="c", subcore_axis_name="s")

    @functools.partial(
        pl.kernel,
        mesh=mesh,
        out_type=jax.ShapeDtypeStruct((N, H), jnp.float32),
        scratch_types=[
            pltpu.VMEM((BPW,), jnp.int32),
            pltpu.VMEM((BPW, H), jnp.float32),
            pltpu.SemaphoreType.DMA,
        ],
    )
    def scatter(y_hbm, idx_hbm, out_hbm, idx_v, rows_v, sem):
        wid = lax.axis_index("s") * 2 + lax.axis_index("c")
        base = wid * BPW
        pltpu.sync_copy(idx_hbm.at[pl.ds(base, BPW)], idx_v)
        pltpu.sync_copy(y_hbm.at[pl.ds(base, BPW)], rows_v)
        pltpu.async_copy(rows_v, out_hbm.at[idx_v], sem).wait()

    return scatter


def _gmm_body(tb_ref, eb_ref, lo_ref, hi_ref, first_ref,
              x_ref, wg_ref, wu_ref, wd_ref, out_ref):
    w = pl.program_id(0)

    @pl.when(first_ref[w] == 1)
    def _():
        out_ref[...] = jnp.zeros_like(out_ref)

    lo = lo_ref[w]
    hi = hi_ref[w]

    @pl.when(hi > lo)
    def _():
        rows = lax.broadcasted_iota(jnp.int32, (B_TOK, 1), 0)
        msk = ((rows >= lo) & (rows < hi)).astype(jnp.float32)
        xm = x_ref[...] * msk
        g = lax.dot_general(xm, wg_ref[0], (((1,), (1,)), ((), ())),
                            preferred_element_type=jnp.float32)
        u = lax.dot_general(xm, wu_ref[0], (((1,), (1,)), ((), ())),
                            preferred_element_type=jnp.float32)
        hdn = (g * jax.nn.sigmoid(g)) * u
        y = lax.dot_general(hdn, wd_ref[0], (((1,), (1,)), ((), ())),
                            preferred_element_type=jnp.float32)
        out_ref[...] += y


def _gmm(x_sorted, Wg, Wu, Wd, tb, eb, lo, hi, first):
    grid_spec = pltpu.PrefetchScalarGridSpec(
        num_scalar_prefetch=5,
        grid=(G,),
        in_specs=[
            pl.BlockSpec((B_TOK, H), lambda w, tb, eb, lo, hi, first: (tb[w], 0)),
            pl.BlockSpec((1, F, H), lambda w, tb, eb, lo, hi, first: (eb[w], 0, 0)),
            pl.BlockSpec((1, F, H), lambda w, tb, eb, lo, hi, first: (eb[w], 0, 0)),
            pl.BlockSpec((1, H, F), lambda w, tb, eb, lo, hi, first: (eb[w], 0, 0)),
        ],
        out_specs=pl.BlockSpec((B_TOK, H), lambda w, tb, eb, lo, hi, first: (tb[w], 0)),
    )
    return pl.pallas_call(
        _gmm_body,
        grid_spec=grid_spec,
        out_shape=jax.ShapeDtypeStruct((N, H), jnp.float32),
        compiler_params=pltpu.CompilerParams(
            dimension_semantics=("arbitrary",)),
    )(tb, eb, lo, hi, first, x_sorted, Wg, Wu, Wd)


def _schedule(counts):
    ends = jnp.cumsum(counts)
    starts = ends - counts
    nonempty = counts > 0
    fb = jnp.where(nonempty, starts // B_TOK, 0)
    lb = jnp.where(nonempty, (ends - 1) // B_TOK, -1)
    nb = jnp.where(nonempty, lb - fb + 1, 0).astype(jnp.int32)
    cnb = jnp.cumsum(nb)
    total = cnb[E - 1]
    w = jnp.arange(G, dtype=jnp.int32)
    wc = jnp.minimum(w, total - 1)
    e_of = jnp.searchsorted(cnb, wc, side="right").astype(jnp.int32)
    ws = cnb[e_of] - nb[e_of]
    tb = (fb[e_of] + (wc - ws)).astype(jnp.int32)
    lo = jnp.maximum(starts[e_of] - tb * B_TOK, 0)
    hi = jnp.minimum(ends[e_of] - tb * B_TOK, B_TOK)
    valid = w < total
    lo = jnp.where(valid, lo, 0).astype(jnp.int32)
    hi = jnp.where(valid, hi, 0).astype(jnp.int32)
    first = jnp.concatenate(
        [jnp.ones((1,), jnp.int32), (tb[1:] != tb[:-1]).astype(jnp.int32)])
    return tb, e_of, lo, hi, first


def kernel(x, Wr, Wg, Wu, Wd):
    b, s, d = x.shape
    x_flat = x.reshape(N, H)
    eid, counts = _router(x_flat, Wr)
    sort_idx = jnp.argsort(eid).astype(jnp.int32)
    tb, eb, lo, hi, first = _schedule(counts)
    x_sorted = _sc_gather_kernel()(x_flat, sort_idx)
    y_sorted = _gmm(x_sorted, Wg, Wu, Wd, tb, eb, lo, hi, first)
    out = _sc_scatter_kernel()(y_sorted, sort_idx)
    return out.reshape(b, s, d)
